# Initial kernel scaffold; baseline (speedup 1.0000x reference)
import jax
import jax.numpy as jnp
from jax.experimental import pallas as pl


def kernel(env_ids, table):
    raise NotImplementedError("write your pallas kernel here")


# speedup vs baseline: 2.7546x; 2.7546x over previous
import functools
import jax
import jax.numpy as jnp
from jax import lax
from jax.experimental import pallas as pl
from jax.experimental.pallas import tpu as pltpu
from jax.experimental.pallas import tpu_sc as plsc

NUM_ROWS = 1000
D = 128
B_TOTAL = 4096 * 50
NC, NS = 2, 16
NW = NC * NS
B_PER_W = B_TOTAL // NW
CHUNK = 128
NCHUNK = B_PER_W // CHUNK


def _make_kernel():
  mesh = plsc.VectorSubcoreMesh(core_axis_name="c", subcore_axis_name="s")

  @functools.partial(
      pl.kernel,
      out_type=jax.ShapeDtypeStruct((B_TOTAL, D), jnp.float32),
      mesh=mesh,
      scratch_types=[
          pltpu.VMEM((NCHUNK, CHUNK), jnp.int32),
          pltpu.VMEM((CHUNK, D), jnp.float32),
          pltpu.SemaphoreType.DMA,
      ],
  )
  def gather_kernel(idx_hbm, table_hbm, out_hbm, idx_v, rows_v, sem):
    wid = lax.axis_index("s") * NC + lax.axis_index("c")
    pltpu.sync_copy(idx_hbm.at[wid], idx_v)

    def body(j, carry):
      pltpu.async_copy(table_hbm.at[idx_v.at[j]], rows_v, sem).wait()
      pltpu.sync_copy(rows_v, out_hbm.at[pl.ds(wid * B_PER_W + j * CHUNK, CHUNK)])
      return carry

    lax.fori_loop(0, NCHUNK, body, 0)

  return gather_kernel


_gather = _make_kernel()


@jax.jit
def kernel(env_ids, table):
  idx = env_ids.reshape(NW, NCHUNK, CHUNK).astype(jnp.int32)
  out = _gather(idx, table)
  return out.reshape(env_ids.shape + (D,))


# speedup vs baseline: 2.8944x; 1.0508x over previous
import functools
import jax
import jax.numpy as jnp
from jax import lax
from jax.experimental import pallas as pl
from jax.experimental.pallas import tpu as pltpu
from jax.experimental.pallas import tpu_sc as plsc

NUM_ROWS = 1000
D = 128
B_TOTAL = 4096 * 50
NC, NS = 2, 16
NW = NC * NS
B_PER_W = B_TOTAL // NW
CHUNK = 128
NCHUNK = B_PER_W // CHUNK


def _make_kernel():
  mesh = plsc.VectorSubcoreMesh(core_axis_name="c", subcore_axis_name="s")

  @functools.partial(
      pl.kernel,
      out_type=jax.ShapeDtypeStruct((B_TOTAL, D), jnp.float32),
      mesh=mesh,
      scratch_types=[
          pltpu.VMEM((NCHUNK, CHUNK), jnp.int32),
          pltpu.VMEM((2, CHUNK, D), jnp.float32),
          pltpu.SemaphoreType.DMA,
          pltpu.SemaphoreType.DMA,
      ],
  )
  def gather_kernel(idx_hbm, table_hbm, out_hbm, idx_v, rows_v, sem_g, sem_s):
    wid = lax.axis_index("s") * NC + lax.axis_index("c")
    base = wid * B_PER_W
    pltpu.sync_copy(idx_hbm.at[wid], idx_v)

    pltpu.async_copy(table_hbm.at[idx_v.at[0]], rows_v.at[0], sem_g)

    def body(j, carry):
      p = lax.rem(j, 2)
      q = 1 - p

      @pl.when(j >= 1)
      def _wait_prev_scatter():
        pltpu.make_async_copy(
            rows_v.at[q], out_hbm.at[pl.ds(base, CHUNK)], sem_s).wait()

      @pl.when(j + 1 < NCHUNK)
      def _fire_next_gather():
        pltpu.async_copy(table_hbm.at[idx_v.at[j + 1]], rows_v.at[q], sem_g)

      pltpu.make_async_copy(
          table_hbm.at[idx_v.at[j]], rows_v.at[p], sem_g).wait()
      pltpu.async_copy(
          rows_v.at[p], out_hbm.at[pl.ds(base + j * CHUNK, CHUNK)], sem_s)
      return carry

    lax.fori_loop(0, NCHUNK, body, 0)
    pltpu.make_async_copy(
        rows_v.at[0], out_hbm.at[pl.ds(base, CHUNK)], sem_s).wait()

  return gather_kernel


_gather = _make_kernel()


@jax.jit
def kernel(env_ids, table):
  idx = env_ids.reshape(NW, NCHUNK, CHUNK).astype(jnp.int32)
  out = _gather(idx, table)
  return out.reshape(env_ids.shape + (D,))


# speedup vs baseline: 3.6962x; 1.2770x over previous
import functools
import jax
import jax.numpy as jnp
from jax import lax
from jax.experimental import pallas as pl
from jax.experimental.pallas import tpu as pltpu
from jax.experimental.pallas import tpu_sc as plsc

NUM_ROWS = 1000
D = 128
B_TOTAL = 4096 * 50
NC, NS = 2, 16
NW = NC * NS
B_PER_W = B_TOTAL // NW
CHUNK = 128
NCHUNK = B_PER_W // CHUNK


def _make_kernel():
  mesh = plsc.VectorSubcoreMesh(core_axis_name="c", subcore_axis_name="s")

  @functools.partial(
      pl.kernel,
      out_type=jax.ShapeDtypeStruct((B_TOTAL, D), jnp.float32),
      mesh=mesh,
      scratch_types=[
          pltpu.VMEM((NCHUNK, CHUNK), jnp.int32),
          pltpu.VMEM((2, CHUNK, D), jnp.float32),
          pltpu.VMEM_SHARED((NUM_ROWS, D), jnp.float32),
          pltpu.SemaphoreType.DMA,
          pltpu.SemaphoreType.DMA,
      ],
  )
  def gather_kernel(idx_hbm, table_hbm, out_hbm, idx_v, rows_v, table_sh,
                    sem_g, sem_s):
    sid = lax.axis_index("s")
    wid = sid * NC + lax.axis_index("c")
    base = wid * B_PER_W
    pltpu.sync_copy(idx_hbm.at[wid], idx_v)

    @pl.when(sid == 0)
    def _stage_table():
      pltpu.sync_copy(table_hbm, table_sh)

    plsc.subcore_barrier()

    pltpu.async_copy(table_sh.at[idx_v.at[0]], rows_v.at[0], sem_g)

    def body(j, carry):
      p = lax.rem(j, 2)
      q = 1 - p

      @pl.when(j >= 1)
      def _wait_prev_scatter():
        pltpu.make_async_copy(
            rows_v.at[q], out_hbm.at[pl.ds(base, CHUNK)], sem_s).wait()

      @pl.when(j + 1 < NCHUNK)
      def _fire_next_gather():
        pltpu.async_copy(table_sh.at[idx_v.at[j + 1]], rows_v.at[q], sem_g)

      pltpu.make_async_copy(
          table_sh.at[idx_v.at[j]], rows_v.at[p], sem_g).wait()
      pltpu.async_copy(
          rows_v.at[p], out_hbm.at[pl.ds(base + j * CHUNK, CHUNK)], sem_s)
      return carry

    lax.fori_loop(0, NCHUNK, body, 0)
    pltpu.make_async_copy(
        rows_v.at[0], out_hbm.at[pl.ds(base, CHUNK)], sem_s).wait()

  return gather_kernel


_gather = _make_kernel()


@jax.jit
def kernel(env_ids, table):
  idx = env_ids.reshape(NW, NCHUNK, CHUNK).astype(jnp.int32)
  out = _gather(idx, table)
  return out.reshape(env_ids.shape + (D,))


# speedup vs baseline: 7.4041x; 2.0032x over previous
import functools
import jax
import jax.numpy as jnp
from jax import lax
from jax.experimental import pallas as pl
from jax.experimental.pallas import tpu as pltpu
from jax.experimental.pallas import tpu_sc as plsc

NUM_ROWS = 1000
D = 128
NSEQ = 4096
SEQ_LEN = 50
NC, NS = 2, 16
NW = NC * NS
S_PER_W = NSEQ // NW


def _make_kernel():
  mesh = plsc.VectorSubcoreMesh(core_axis_name="c", subcore_axis_name="s")

  @functools.partial(
      pl.kernel,
      out_type=jax.ShapeDtypeStruct((NSEQ, SEQ_LEN, D), jnp.float32),
      mesh=mesh,
      compiler_params=pltpu.CompilerParams(use_tc_tiling_on_sc=True),
      scratch_types=[
          pltpu.VMEM((S_PER_W, SEQ_LEN), jnp.int32),
          pltpu.VMEM((2, SEQ_LEN, D), jnp.float32),
          pltpu.VMEM_SHARED((NUM_ROWS, D), jnp.float32),
          pltpu.SemaphoreType.DMA,
          pltpu.SemaphoreType.DMA,
      ],
  )
  def gather_kernel(idx_hbm, table_hbm, out_hbm, idx_v, rows_v, table_sh,
                    sem_g, sem_s):
    sid = lax.axis_index("s")
    wid = sid * NC + lax.axis_index("c")
    base = wid * S_PER_W
    pltpu.sync_copy(idx_hbm.at[pl.ds(base, S_PER_W)], idx_v)

    @pl.when(sid == 0)
    def _stage_table():
      pltpu.sync_copy(table_hbm, table_sh)

    plsc.subcore_barrier()

    pltpu.async_copy(table_sh.at[idx_v.at[0]], rows_v.at[0], sem_g)

    def body(s, carry):
      p = lax.rem(s, 2)
      q = 1 - p

      @pl.when(s >= 1)
      def _wait_prev_scatter():
        pltpu.make_async_copy(rows_v.at[q], out_hbm.at[base], sem_s).wait()

      @pl.when(s + 1 < S_PER_W)
      def _fire_next_gather():
        pltpu.async_copy(table_sh.at[idx_v.at[s + 1]], rows_v.at[q], sem_g)

      pltpu.make_async_copy(
          table_sh.at[idx_v.at[s]], rows_v.at[p], sem_g).wait()
      pltpu.async_copy(rows_v.at[p], out_hbm.at[base + s], sem_s)
      return carry

    lax.fori_loop(0, S_PER_W, body, 0)
    pltpu.make_async_copy(rows_v.at[0], out_hbm.at[base], sem_s).wait()

  return gather_kernel


_gather = _make_kernel()


@jax.jit
def kernel(env_ids, table):
  return _gather(env_ids.astype(jnp.int32), table)
